# baseline (speedup 1.0000x reference)
import functools

import jax
import jax.numpy as jnp
from jax import lax
from jax.experimental import pallas as pl
from jax.experimental.pallas import tpu as pltpu
from jax.experimental.pallas import tpu_sc as plsc

NC = 2
NS = 16
NW = NC * NS
NBUF = 4
DP = 128
BL = 128


def _make_sc_gather(nb, h, d):
  bpw = nb // NW
  blk_pw = bpw // BL
  n_ch = blk_pw * h
  mesh = plsc.VectorSubcoreMesh(core_axis_name="c", subcore_axis_name="s")

  @functools.partial(
      pl.kernel,
      mesh=mesh,
      compiler_params=pltpu.CompilerParams(
          use_tc_tiling_on_sc=True, needs_layout_passes=False),
      out_type=jax.ShapeDtypeStruct((h, d, nb), jnp.float32),
      scratch_types=[
          pltpu.VMEM((h, bpw), jnp.int32),
          pltpu.VMEM((NBUF, BL, DP), jnp.float32),
          pltpu.VMEM((NBUF, d, BL), jnp.float32),
          pltpu.SemaphoreType.DMA((NBUF,)),
          pltpu.SemaphoreType.DMA((NBUF,)),
      ],
  )
  def sc_gather(idx_hbm, table_hbm, out_hbm, idx_v, g_v, t_v, gsem, osem):
    wid = lax.axis_index("s") * NC + lax.axis_index("c")
    pltpu.sync_copy(idx_hbm.at[wid], idx_v)

    iota = lax.broadcasted_iota(jnp.int32, (16,), 0)
    dvecs = [iota + (g * 16) for g in range(d // 16)]

    def coords(c):
      o = c // h
      hh = c - o * h
      return hh, o

    def g_start(c, b):
      hh, o = coords(c)
      pltpu.async_copy(
          table_hbm.at[idx_v.at[hh, pl.ds(o * BL, BL)]], g_v.at[b],
          gsem.at[b])

    def g_wait(c, b):
      hh, o = coords(c)
      pltpu.make_async_copy(
          table_hbm.at[idx_v.at[hh, pl.ds(o * BL, BL)]], g_v.at[b],
          gsem.at[b]).wait()

    def out_slice(c):
      hh, o = coords(c)
      return out_hbm.at[hh, :, pl.ds((wid * blk_pw + o) * BL, BL)]

    def w_start(c, b):
      pltpu.async_copy(t_v.at[b], out_slice(c), osem.at[b])

    def w_wait(c, b):
      pltpu.make_async_copy(t_v.at[b], out_slice(c), osem.at[b]).wait()

    def transpose(b):
      def tr_body(bl, _):
        blv = jnp.full((16,), 0, jnp.int32) + bl
        for g in range(d // 16):
          vals = plsc.load_gather(g_v.at[b], [blv, dvecs[g]])
          plsc.store_scatter(t_v.at[b], [dvecs[g], blv], vals)
        return ()
      lax.fori_loop(0, BL, tr_body, (), unroll=8)

    for b in range(NBUF):
      g_start(b, b)
    for b in range(NBUF):
      g_wait(b, b)
      transpose(b)
      g_start(b + NBUF, b)
      w_start(b, b)

    def body(i, _):
      for b in range(NBUF):
        c = NBUF + i * NBUF + b
        g_wait(c, b)
        w_wait(c - NBUF, b)
        transpose(b)
        g_start(c + NBUF, b)
        w_start(c, b)
      return ()

    lax.fori_loop(0, (n_ch - 2 * NBUF) // NBUF, body, (), unroll=False)

    for b in range(NBUF):
      c = n_ch - NBUF + b
      g_wait(c, b)
      w_wait(c - NBUF, b)
      transpose(b)
      w_start(c, b)
    for b in range(NBUF):
      w_wait(n_ch - NBUF + b, b)

  return sc_gather


def kernel(inp, table):
  nb, h = inp.shape
  v, d = table.shape
  assert nb % (NW * BL) == 0 and d % 16 == 0
  table_p = jnp.pad(table, ((0, 0), (0, DP - d)))
  idx = jnp.swapaxes(inp, 0, 1).reshape(h, NW, nb // NW).transpose(1, 0, 2)
  idx = idx.astype(jnp.int32)
  fn = _make_sc_gather(nb, h, d)
  out = fn(idx, table_p)
  return jnp.transpose(out, (2, 0, 1))


# speedup vs baseline: 1.1400x; 1.1400x over previous
import functools

import jax
import jax.numpy as jnp
from jax import lax
from jax.experimental import pallas as pl
from jax.experimental.pallas import tpu as pltpu
from jax.experimental.pallas import tpu_sc as plsc

NC = 2
NS = 16
NW = NC * NS
NBUF = 8
DP = 128


def _make_sc_gather(nb, h, d):
  rows_pw = nb // NW
  mesh = plsc.VectorSubcoreMesh(core_axis_name="c", subcore_axis_name="s")

  @functools.partial(
      pl.kernel,
      mesh=mesh,
      compiler_params=pltpu.CompilerParams(use_tc_tiling_on_sc=False),
      out_type=jax.ShapeDtypeStruct((nb, h, DP), jnp.float32),
      scratch_types=[
          pltpu.VMEM((rows_pw, h), jnp.int32),
          pltpu.VMEM((NBUF, h, d), jnp.float32),
          pltpu.SemaphoreType.DMA((NBUF,)),
          pltpu.SemaphoreType.DMA((NBUF,)),
      ],
  )
  def sc_gather(idx_hbm, table_hbm, out_hbm, idx_v, rows_v, gsem, osem):
    wid = lax.axis_index("s") * NC + lax.axis_index("c")
    base = wid * rows_pw
    pltpu.sync_copy(idx_hbm.at[wid], idx_v)

    def g_start(j, b):
      pltpu.async_copy(table_hbm.at[idx_v.at[j]], rows_v.at[b], gsem.at[b])

    def g_wait(j, b):
      pltpu.make_async_copy(
          table_hbm.at[idx_v.at[j]], rows_v.at[b], gsem.at[b]).wait()

    def w_start(j, b):
      pltpu.async_copy(
          rows_v.at[b], out_hbm.at[base + j, :, pl.ds(0, d)], osem.at[b])

    def w_wait(j, b):
      pltpu.make_async_copy(
          rows_v.at[b], out_hbm.at[base + j, :, pl.ds(0, d)],
          osem.at[b]).wait()

    for b in range(NBUF):
      g_start(b, b)

    def body(j0, _):
      for b in range(NBUF):
        j = j0 + b
        g_wait(j, b)
        w_start(j, b)
        w_wait(j, b)
        g_start(j + NBUF, b)
      return ()

    lax.fori_loop(0, (rows_pw - NBUF) // NBUF,
                  lambda i, c: body(i * NBUF, c), (), unroll=False)

    for b in range(NBUF):
      j = rows_pw - NBUF + b
      g_wait(j, b)
      w_start(j, b)
    for b in range(NBUF):
      j = rows_pw - NBUF + b
      w_wait(j, b)

  return sc_gather


def kernel(inp, table):
  nb, h = inp.shape
  v, d = table.shape
  assert nb % (NW * NBUF) == 0
  idx = inp.reshape(NW, nb // NW, h).astype(jnp.int32)
  fn = _make_sc_gather(nb, h, d)
  return fn(idx, table)[:, :, :d]


# speedup vs baseline: 1.5270x; 1.3394x over previous
import functools

import jax
import jax.numpy as jnp
from jax import lax
from jax.experimental import pallas as pl
from jax.experimental.pallas import tpu as pltpu
from jax.experimental.pallas import tpu_sc as plsc

NC = 2
NS = 16
NW = NC * NS
NBUF = 8
DP = 128


def _make_sc_gather(nb, h, d):
  rows_pw = nb // NW
  mesh = plsc.VectorSubcoreMesh(core_axis_name="c", subcore_axis_name="s")

  @functools.partial(
      pl.kernel,
      mesh=mesh,
      compiler_params=pltpu.CompilerParams(use_tc_tiling_on_sc=True),
      out_type=jax.ShapeDtypeStruct((nb, h, DP), jnp.float32),
      scratch_types=[
          pltpu.VMEM((rows_pw, h), jnp.int32),
          pltpu.VMEM((NBUF, h, DP), jnp.float32),
          pltpu.SemaphoreType.DMA((NBUF,)),
          pltpu.SemaphoreType.DMA((NBUF,)),
      ],
  )
  def sc_gather(idx_hbm, table_hbm, out_hbm, idx_v, rows_v, gsem, osem):
    wid = lax.axis_index("s") * NC + lax.axis_index("c")
    base = wid * rows_pw
    pltpu.sync_copy(idx_hbm.at[wid], idx_v)

    def g_start(j, b):
      pltpu.async_copy(table_hbm.at[idx_v.at[j]], rows_v.at[b], gsem.at[b])

    def g_wait(j, b):
      pltpu.make_async_copy(
          table_hbm.at[idx_v.at[j]], rows_v.at[b], gsem.at[b]).wait()

    def w_start(j, b):
      pltpu.async_copy(rows_v.at[b], out_hbm.at[base + j], osem.at[b])

    def w_wait(j, b):
      pltpu.make_async_copy(
          rows_v.at[b], out_hbm.at[base + j], osem.at[b]).wait()

    for b in range(NBUF):
      g_start(b, b)

    def body(j0, _):
      for b in range(NBUF):
        j = j0 + b
        g_wait(j, b)
        w_start(j, b)
        w_wait(j, b)
        g_start(j + NBUF, b)
      return ()

    lax.fori_loop(0, (rows_pw - NBUF) // NBUF,
                  lambda i, c: body(i * NBUF, c), (), unroll=False)

    for b in range(NBUF):
      j = rows_pw - NBUF + b
      g_wait(j, b)
      w_start(j, b)
    for b in range(NBUF):
      j = rows_pw - NBUF + b
      w_wait(j, b)

  return sc_gather


def kernel(inp, table):
  nb, h = inp.shape
  v, d = table.shape
  assert nb % (NW * NBUF) == 0
  table_p = jnp.pad(table, ((0, 0), (0, DP - d)))
  idx = inp.reshape(NW, nb // NW, h).astype(jnp.int32)
  fn = _make_sc_gather(nb, h, d)
  return fn(idx, table_p)[:, :, :d]
